# baseline (speedup 1.0000x reference)
import functools
import jax
import jax.numpy as jnp
from jax import lax
from jax.experimental import pallas as pl
from jax.experimental.pallas import tpu as pltpu
from jax.experimental.pallas import tpu_sc as plsc

VOCAB = 100000
EMB = 128

NC = 2
NS = 16
NW = NC * NS

CHUNK = 128
NBUF = 7


def _make_kernel(n_rows: int):
    assert n_rows % (NW * CHUNK) == 0
    rows_per_w = n_rows // NW
    n_chunks = rows_per_w // CHUNK
    n_groups = n_chunks // NBUF
    n_tail = n_chunks - n_groups * NBUF

    mesh = plsc.VectorSubcoreMesh(core_axis_name="c", subcore_axis_name="s")

    scratch = (
        [pltpu.VMEM((NBUF, CHUNK), jnp.int32)]
        + [pltpu.VMEM((CHUNK, EMB), jnp.float32) for _ in range(NBUF)]
        + [pltpu.SemaphoreType.DMA for _ in range(3 * NBUF)]
    )

    @functools.partial(
        pl.kernel,
        out_type=jax.ShapeDtypeStruct((n_rows, EMB), jnp.float32),
        mesh=mesh,
        scratch_types=scratch,
    )
    def gather_kernel(idx_hbm, table_hbm, out_hbm, idx_v, *rest):
        bufs = rest[:NBUF]
        gsem = rest[NBUF : 2 * NBUF]
        ssem = rest[2 * NBUF : 3 * NBUF]
        isem = rest[3 * NBUF : 4 * NBUF]

        wid = lax.axis_index("s") * NC + lax.axis_index("c")
        row_base = wid * rows_per_w

        def start_idx_load(b, c):
            pltpu.async_copy(idx_hbm.at[wid, c], idx_v.at[b], isem[b])

        def wait_idx_load(b):
            pltpu.make_async_copy(
                idx_hbm.at[wid, 0], idx_v.at[b], isem[b]
            ).wait()

        def start_gather(b, c):
            del c
            pltpu.async_copy(table_hbm.at[idx_v.at[b]], bufs[b], gsem[b])

        def start_scatter(b, c):
            dst = out_hbm.at[pl.ds(row_base + c * CHUNK, CHUNK)]
            pltpu.async_copy(bufs[b], dst, ssem[b])

        def wait_gather(b):
            pltpu.make_async_copy(table_hbm.at[idx_v.at[0]], bufs[b], gsem[b]).wait()

        def wait_scatter(b):
            dst = out_hbm.at[pl.ds(row_base, CHUNK)]
            pltpu.make_async_copy(bufs[b], dst, ssem[b]).wait()

        for b in range(NBUF):
            start_idx_load(b, b)
        for b in range(NBUF - 1):
            wait_idx_load(b)
            start_gather(b, b)

        def step(c, b, q,
                 has_prev_scatter=None, has_next_gather=None,
                 has_idx_prefetch=None):
            def do_all():
                wait_gather(b)

                def idx_prefetch():
                    start_idx_load(b, c + NBUF)

                if has_idx_prefetch is None:
                    pl.when(c + NBUF < n_chunks)(idx_prefetch)
                elif has_idx_prefetch:
                    idx_prefetch()

                start_scatter(b, c)

                def prev_scatter():
                    wait_scatter(q)

                if has_prev_scatter is None:
                    pl.when(c > 0)(prev_scatter)
                elif has_prev_scatter:
                    prev_scatter()

                def next_gather():
                    wait_idx_load(q)
                    start_gather(q, c + NBUF - 1)

                if has_next_gather is None:
                    pl.when(c + NBUF - 1 < n_chunks)(next_gather)
                elif has_next_gather:
                    next_gather()

            do_all()

        def group_body(g, carry):
            for b in range(NBUF):
                c = g * NBUF + b
                step(c, b, (b - 1) % NBUF)
            return carry

        lax.fori_loop(0, n_groups, group_body, 0)

        for t in range(n_tail):
            c = n_groups * NBUF + t
            step(
                c,
                c % NBUF,
                (c - 1) % NBUF,
                has_prev_scatter=True,
                has_next_gather=(c + NBUF - 1 < n_chunks),
                has_idx_prefetch=(c + NBUF < n_chunks),
            )

        wait_scatter((n_chunks - 1) % NBUF)

    return gather_kernel


@jax.jit
def kernel(input_seqs, input_lengths, table):
    del input_lengths
    n_rows = input_seqs.shape[0] * input_seqs.shape[1]
    idx3 = input_seqs.reshape(NW, n_rows // (NW * CHUNK), CHUNK)
    out = _make_kernel(n_rows)(idx3, table)
    return out.reshape(input_seqs.shape[0], input_seqs.shape[1], EMB)
